# baseline (speedup 1.0000x reference)
import functools as _ft

import jax
import jax.numpy as jnp
from jax.experimental import pallas as pl
from jax.experimental.pallas import tpu as pltpu

_BM = 256
_BM2 = 512
_C2 = 1


def _xw_kernel(x_ref, w_ref, b_ref, o_ref):
    o_ref[...] = (
        jnp.dot(x_ref[...].astype(jnp.bfloat16), w_ref[...], preferred_element_type=jnp.float32)
        + b_ref[...]
    ).astype(jnp.bfloat16)


def _layer1_kernel(n, bm, a_ref, h_ref, w_ref, b_ref, o_ref, abf_ref):
    nc = a_ref.shape[1]
    a_bf = a_ref[...].astype(jnp.bfloat16)
    np_cols = abf_ref.shape[1]
    pad = jnp.zeros((bm, np_cols - nc), dtype=jnp.bfloat16)
    abf_ref[...] = jnp.concatenate([a_bf, pad], axis=1)
    acc = jnp.dot(a_bf, h_ref[...], preferred_element_type=jnp.float32)
    acc = jnp.maximum(acc, 0.0).astype(jnp.bfloat16)
    val = (jnp.dot(acc, w_ref[...], preferred_element_type=jnp.float32) + b_ref[...])
    rows = pl.program_id(0) * bm + jax.lax.broadcasted_iota(jnp.int32, val.shape, 0)
    o_ref[...] = jnp.where(rows < n, val, 0.0).astype(jnp.bfloat16)


def _acc_chunks(a_refs, h_ref):
    nc = a_refs[0].shape[1]
    acc = jnp.dot(a_refs[0][...], h_ref[pl.ds(0, nc), :], preferred_element_type=jnp.float32)
    for c in range(1, len(a_refs)):
        acc += jnp.dot(
            a_refs[c][...], h_ref[pl.ds(c * nc, nc), :], preferred_element_type=jnp.float32
        )
    return acc


def _mid_kernel(n, bm, nchunks, *refs):
    a_refs = refs[:nchunks]
    h_ref, w_ref, b_ref, o_ref = refs[nchunks:]
    acc = _acc_chunks(a_refs, h_ref)
    acc = jnp.maximum(acc, 0.0).astype(jnp.bfloat16)
    val = jnp.dot(acc, w_ref[...], preferred_element_type=jnp.float32) + b_ref[...]
    rows = pl.program_id(0) * bm + jax.lax.broadcasted_iota(jnp.int32, val.shape, 0)
    o_ref[...] = jnp.where(rows < n, val, 0.0).astype(jnp.bfloat16)


def _final_kernel(nchunks, *refs):
    a_refs = refs[:nchunks]
    h_ref, o_ref = refs[nchunks:]
    o_ref[...] = _acc_chunks(a_refs, h_ref)


def kernel(x, adj_t, W1, b1, W2, b2, W3, b3):
    n, d_in = x.shape
    d_hid = W1.shape[1]
    d_out = W3.shape[1]
    bm = min(_BM, n)
    nblk = pl.cdiv(n, bm)
    grid = (nblk,)
    n_pad = nblk * bm
    npc = n_pad
    lane = npc // _C2
    bm2 = min(_BM2, n_pad)
    while n_pad % bm2:
        bm2 //= 2
    nblk2 = n_pad // bm2
    grid2 = (nblk2,)

    b1r = b1.reshape(1, -1)
    b2r = b2.reshape(1, -1)
    b3r = b3.reshape(1, -1)
    w1b = W1.astype(jnp.bfloat16)
    w2b = W2.astype(jnp.bfloat16)
    w3b = W3.astype(jnp.bfloat16)

    p1 = pl.pallas_call(
        _xw_kernel,
        grid=grid,
        in_specs=[
            pl.BlockSpec((bm, d_in), lambda i: (i, 0)),
            pl.BlockSpec((d_in, d_hid), lambda i: (0, 0)),
            pl.BlockSpec((1, d_hid), lambda i: (0, 0)),
        ],
        out_specs=pl.BlockSpec((bm, d_hid), lambda i: (i, 0)),
        out_shape=jax.ShapeDtypeStruct((n, d_hid), jnp.bfloat16),
    )(x, w1b, b1r)

    h2, a_bf = pl.pallas_call(
        _ft.partial(_layer1_kernel, n, bm),
        grid=grid,
        in_specs=[
            pl.BlockSpec((bm, n), lambda i: (i, 0)),
            pl.BlockSpec((n, d_hid), lambda i: (0, 0)),
            pl.BlockSpec((d_hid, d_hid), lambda i: (0, 0)),
            pl.BlockSpec((1, d_hid), lambda i: (0, 0)),
        ],
        out_specs=[
            pl.BlockSpec((bm, d_hid), lambda i: (i, 0)),
            pl.BlockSpec((bm, npc), lambda i: (i, 0)),
        ],
        out_shape=[
            jax.ShapeDtypeStruct((n_pad, d_hid), jnp.bfloat16),
            jax.ShapeDtypeStruct((n_pad, npc), jnp.bfloat16),
        ],
    )(adj_t, p1, w2b, b2r)

    a_specs = [
        pl.BlockSpec((bm2, lane), lambda i, c=c: (i, c)) for c in range(_C2)
    ]

    h3 = pl.pallas_call(
        _ft.partial(_mid_kernel, n, bm2, _C2),
        grid=grid2,
        in_specs=a_specs
        + [
            pl.BlockSpec((npc, d_hid), lambda i: (0, 0)),
            pl.BlockSpec((d_hid, d_out), lambda i: (0, 0)),
            pl.BlockSpec((1, d_out), lambda i: (0, 0)),
        ],
        out_specs=pl.BlockSpec((bm2, d_out), lambda i: (i, 0)),
        out_shape=jax.ShapeDtypeStruct((n_pad, d_out), jnp.bfloat16),
    )(*([a_bf] * _C2), h2, w3b, b3r)

    out = pl.pallas_call(
        _ft.partial(_final_kernel, _C2),
        grid=grid2,
        in_specs=a_specs + [pl.BlockSpec((npc, d_out), lambda i: (0, 0))],
        out_specs=pl.BlockSpec((bm2, d_out), lambda i: (i, 0)),
        out_shape=jax.ShapeDtypeStruct((n, d_out), jnp.float32),
    )(*([a_bf] * _C2), h3)

    return out


# speedup vs baseline: 1.0203x; 1.0203x over previous
import functools as _ft

import jax
import jax.numpy as jnp
from jax.experimental import pallas as pl
from jax.experimental.pallas import tpu as pltpu

_BM = 256
_BM2 = 1024
_C2 = 1


def _xw_kernel(x_ref, w_ref, b_ref, o_ref):
    o_ref[...] = (
        jnp.dot(x_ref[...].astype(jnp.bfloat16), w_ref[...], preferred_element_type=jnp.float32)
        + b_ref[...]
    ).astype(jnp.bfloat16)


def _layer1_kernel(n, bm, a_ref, h_ref, w_ref, b_ref, o_ref, abf_ref):
    nc = a_ref.shape[1]
    a_bf = a_ref[...].astype(jnp.bfloat16)
    np_cols = abf_ref.shape[1]
    pad = jnp.zeros((bm, np_cols - nc), dtype=jnp.bfloat16)
    abf_ref[...] = jnp.concatenate([a_bf, pad], axis=1)
    acc = jnp.dot(a_bf, h_ref[...], preferred_element_type=jnp.float32)
    acc = jnp.maximum(acc, 0.0).astype(jnp.bfloat16)
    val = (jnp.dot(acc, w_ref[...], preferred_element_type=jnp.float32) + b_ref[...])
    rows = pl.program_id(0) * bm + jax.lax.broadcasted_iota(jnp.int32, val.shape, 0)
    o_ref[...] = jnp.where(rows < n, val, 0.0).astype(jnp.bfloat16)


def _acc_chunks(a_refs, h_ref):
    nc = a_refs[0].shape[1]
    acc = jnp.dot(a_refs[0][...], h_ref[pl.ds(0, nc), :], preferred_element_type=jnp.float32)
    for c in range(1, len(a_refs)):
        acc += jnp.dot(
            a_refs[c][...], h_ref[pl.ds(c * nc, nc), :], preferred_element_type=jnp.float32
        )
    return acc


def _mid_kernel(n, bm, nchunks, *refs):
    a_refs = refs[:nchunks]
    h_ref, w_ref, b_ref, o_ref = refs[nchunks:]
    acc = _acc_chunks(a_refs, h_ref)
    acc = jnp.maximum(acc, 0.0).astype(jnp.bfloat16)
    val = jnp.dot(acc, w_ref[...], preferred_element_type=jnp.float32) + b_ref[...]
    rows = pl.program_id(0) * bm + jax.lax.broadcasted_iota(jnp.int32, val.shape, 0)
    o_ref[...] = jnp.where(rows < n, val, 0.0).astype(jnp.bfloat16)


def _final_kernel(nchunks, *refs):
    a_refs = refs[:nchunks]
    h_ref, o_ref = refs[nchunks:]
    o_ref[...] = _acc_chunks(a_refs, h_ref)


def kernel(x, adj_t, W1, b1, W2, b2, W3, b3):
    n, d_in = x.shape
    d_hid = W1.shape[1]
    d_out = W3.shape[1]
    bm = min(_BM, n)
    nblk = pl.cdiv(n, bm)
    grid = (nblk,)
    n_pad = nblk * bm
    npc = n_pad
    lane = npc // _C2
    bm2 = min(_BM2, n_pad)
    while n_pad % bm2:
        bm2 //= 2
    nblk2 = n_pad // bm2
    grid2 = (nblk2,)

    b1r = b1.reshape(1, -1)
    b2r = b2.reshape(1, -1)
    b3r = b3.reshape(1, -1)
    w1b = W1.astype(jnp.bfloat16)
    w2b = W2.astype(jnp.bfloat16)
    w3b = W3.astype(jnp.bfloat16)

    p1 = pl.pallas_call(
        _xw_kernel,
        grid=grid,
        in_specs=[
            pl.BlockSpec((bm, d_in), lambda i: (i, 0)),
            pl.BlockSpec((d_in, d_hid), lambda i: (0, 0)),
            pl.BlockSpec((1, d_hid), lambda i: (0, 0)),
        ],
        out_specs=pl.BlockSpec((bm, d_hid), lambda i: (i, 0)),
        out_shape=jax.ShapeDtypeStruct((n, d_hid), jnp.bfloat16),
    )(x, w1b, b1r)

    h2, a_bf = pl.pallas_call(
        _ft.partial(_layer1_kernel, n, bm),
        grid=grid,
        in_specs=[
            pl.BlockSpec((bm, n), lambda i: (i, 0)),
            pl.BlockSpec((n, d_hid), lambda i: (0, 0)),
            pl.BlockSpec((d_hid, d_hid), lambda i: (0, 0)),
            pl.BlockSpec((1, d_hid), lambda i: (0, 0)),
        ],
        out_specs=[
            pl.BlockSpec((bm, d_hid), lambda i: (i, 0)),
            pl.BlockSpec((bm, npc), lambda i: (i, 0)),
        ],
        out_shape=[
            jax.ShapeDtypeStruct((n_pad, d_hid), jnp.bfloat16),
            jax.ShapeDtypeStruct((n_pad, npc), jnp.bfloat16),
        ],
    )(adj_t, p1, w2b, b2r)

    a_specs = [
        pl.BlockSpec((bm2, lane), lambda i, c=c: (i, c)) for c in range(_C2)
    ]

    h3 = pl.pallas_call(
        _ft.partial(_mid_kernel, n, bm2, _C2),
        grid=grid2,
        in_specs=a_specs
        + [
            pl.BlockSpec((npc, d_hid), lambda i: (0, 0)),
            pl.BlockSpec((d_hid, d_out), lambda i: (0, 0)),
            pl.BlockSpec((1, d_out), lambda i: (0, 0)),
        ],
        out_specs=pl.BlockSpec((bm2, d_out), lambda i: (i, 0)),
        out_shape=jax.ShapeDtypeStruct((n_pad, d_out), jnp.bfloat16),
    )(*([a_bf] * _C2), h2, w3b, b3r)

    out = pl.pallas_call(
        _ft.partial(_final_kernel, _C2),
        grid=grid2,
        in_specs=a_specs + [pl.BlockSpec((npc, d_out), lambda i: (0, 0))],
        out_specs=pl.BlockSpec((bm2, d_out), lambda i: (i, 0)),
        out_shape=jax.ShapeDtypeStruct((n, d_out), jnp.float32),
    )(*([a_bf] * _C2), h3)

    return out


# speedup vs baseline: 1.0786x; 1.0572x over previous
import jax
import jax.numpy as jnp
from jax.experimental import pallas as pl
from jax.experimental.pallas import tpu as pltpu

_BM = 400
_BM2 = 1000


def _xw_kernel(x_ref, w_ref, b_ref, o_ref):
    o_ref[...] = (
        jnp.dot(x_ref[...].astype(jnp.bfloat16), w_ref[...], preferred_element_type=jnp.float32)
        + b_ref[...]
    ).astype(jnp.bfloat16)


def _layer1_kernel(a_ref, h_ref, w_ref, b_ref, o_ref, abf_ref):
    a_bf = a_ref[...].astype(jnp.bfloat16)
    abf_ref[...] = a_bf
    acc = jnp.dot(a_bf, h_ref[...], preferred_element_type=jnp.float32)
    acc = jnp.maximum(acc, 0.0).astype(jnp.bfloat16)
    o_ref[...] = (
        jnp.dot(acc, w_ref[...], preferred_element_type=jnp.float32) + b_ref[...]
    ).astype(jnp.bfloat16)


def _mid_kernel(a_ref, h_ref, w_ref, b_ref, o_ref):
    acc = jnp.dot(a_ref[...], h_ref[...], preferred_element_type=jnp.float32)
    acc = jnp.maximum(acc, 0.0).astype(jnp.bfloat16)
    o_ref[...] = (
        jnp.dot(acc, w_ref[...], preferred_element_type=jnp.float32) + b_ref[...]
    ).astype(jnp.bfloat16)


def _final_kernel(a_ref, h_ref, o_ref):
    o_ref[...] = jnp.dot(a_ref[...], h_ref[...], preferred_element_type=jnp.float32)


def _pick_bm(n, want):
    bm = min(want, n)
    while n % bm or bm % 8:
        bm -= 8 if bm % 8 == 0 else bm % 8
        if bm <= 0:
            return n
    return bm


def kernel(x, adj_t, W1, b1, W2, b2, W3, b3):
    n, d_in = x.shape
    d_hid = W1.shape[1]
    d_out = W3.shape[1]
    bm = _pick_bm(n, _BM)
    grid = (n // bm,)
    bm2 = _pick_bm(n, _BM2)
    grid2 = (n // bm2,)

    b1r = b1.reshape(1, -1)
    b2r = b2.reshape(1, -1)
    b3r = b3.reshape(1, -1)
    w1b = W1.astype(jnp.bfloat16)
    w2b = W2.astype(jnp.bfloat16)
    w3b = W3.astype(jnp.bfloat16)

    p1 = pl.pallas_call(
        _xw_kernel,
        grid=grid,
        in_specs=[
            pl.BlockSpec((bm, d_in), lambda i: (i, 0)),
            pl.BlockSpec((d_in, d_hid), lambda i: (0, 0)),
            pl.BlockSpec((1, d_hid), lambda i: (0, 0)),
        ],
        out_specs=pl.BlockSpec((bm, d_hid), lambda i: (i, 0)),
        out_shape=jax.ShapeDtypeStruct((n, d_hid), jnp.bfloat16),
    )(x, w1b, b1r)

    h2, a_bf = pl.pallas_call(
        _layer1_kernel,
        grid=grid,
        in_specs=[
            pl.BlockSpec((bm, n), lambda i: (i, 0)),
            pl.BlockSpec((n, d_hid), lambda i: (0, 0)),
            pl.BlockSpec((d_hid, d_hid), lambda i: (0, 0)),
            pl.BlockSpec((1, d_hid), lambda i: (0, 0)),
        ],
        out_specs=[
            pl.BlockSpec((bm, d_hid), lambda i: (i, 0)),
            pl.BlockSpec((bm, n), lambda i: (i, 0)),
        ],
        out_shape=[
            jax.ShapeDtypeStruct((n, d_hid), jnp.bfloat16),
            jax.ShapeDtypeStruct((n, n), jnp.bfloat16),
        ],
    )(adj_t, p1, w2b, b2r)

    h3 = pl.pallas_call(
        _mid_kernel,
        grid=grid2,
        in_specs=[
            pl.BlockSpec((bm2, n), lambda i: (i, 0)),
            pl.BlockSpec((n, d_hid), lambda i: (0, 0)),
            pl.BlockSpec((d_hid, d_out), lambda i: (0, 0)),
            pl.BlockSpec((1, d_out), lambda i: (0, 0)),
        ],
        out_specs=pl.BlockSpec((bm2, d_out), lambda i: (i, 0)),
        out_shape=jax.ShapeDtypeStruct((n, d_out), jnp.bfloat16),
    )(a_bf, h2, w3b, b3r)

    out = pl.pallas_call(
        _final_kernel,
        grid=grid2,
        in_specs=[
            pl.BlockSpec((bm2, n), lambda i: (i, 0)),
            pl.BlockSpec((n, d_out), lambda i: (0, 0)),
        ],
        out_specs=pl.BlockSpec((bm2, d_out), lambda i: (i, 0)),
        out_shape=jax.ShapeDtypeStruct((n, d_out), jnp.float32),
    )(a_bf, h3)

    return out
